# baseline (speedup 1.0000x reference)
import functools

import jax
import jax.numpy as jnp
from jax import lax
from jax.experimental import pallas as pl
from jax.experimental.pallas import tpu as pltpu
from jax.experimental.pallas import tpu_sc as plsc

NC = 2
NS = 16
L = 16
B = 80


def _sc_body(N, CH, n_echunk, n_fchunk,
             zs, zf, colb_h, rowb_h, outl, outh,
             colb, rowb, gbuf, onesb, degb,
             acc, deg, sem0, sem1):
    cid = lax.axis_index("c")
    sid = lax.axis_index("s")
    zero16 = jnp.zeros((L,), jnp.float32)
    ones16 = jnp.ones((L,), jnp.float32)
    g0 = gbuf.at[pl.ds(0, B)]
    g1 = gbuf.at[pl.ds(B, B)]

    def init_row(r, carry):
        for c8 in range(CH // L):
            gbuf[r, 0, pl.ds(c8 * L, L)] = zero16
        return carry
    lax.fori_loop(0, 2 * B, init_row, 0)

    def init_small(r, carry):
        onesb[pl.ds(r * L, L)] = zero16
        return carry
    lax.fori_loop(0, (B + L) // L, init_small, 0)

    ne = colb.shape[0]
    pltpu.sync_copy(colb_h.at[pl.ds(sid * ne, ne)], colb)
    pltpu.sync_copy(rowb_h.at[sid], rowb)

    def col_fix(i, carry):
        sl = pl.ds(i * L, L)
        colb[sl] = colb[sl] * 2 + cid
        return carry
    lax.fori_loop(0, ne // L, col_fix, 0)

    def zero_chunk(c, carry):
        ch = sid + NS * c
        @pl.when(ch < N // B)
        def _():
            pltpu.sync_copy(g0, acc.at[pl.ds(ch * B, B)])
            pltpu.sync_copy(onesb.at[pl.ds(0, B)], deg.at[pl.ds(ch * B, B)])
        return carry
    lax.fori_loop(0, (N // B + NS - 1) // NS, zero_chunk, 0)

    def ones_row(r, carry):
        onesb[pl.ds(r * L, L)] = ones16
        return carry
    lax.fori_loop(0, (B + L) // L, ones_row, 0)
    plsc.subcore_barrier()

    def start_gather(k, gb, sem):
        pltpu.async_copy(zs.at[colb.at[pl.ds(k * B, B)]], gb, sem)

    def drain_slot(k, gb, sem):
        pltpu.make_async_copy(zs.at[colb.at[pl.ds(k * B, B)]], gb, sem).wait()
        pltpu.sync_copy(gb, acc.at[rowb.at[k]], add=True)
        pltpu.sync_copy(onesb.at[pl.ds(0, B)], deg.at[rowb.at[k]], add=True)

    start_gather(0, g0, sem0)
    n2 = n_echunk // 2

    def edge_pair(g, carry):
        start_gather(2 * g + 1, g1, sem1)
        drain_slot(2 * g, g0, sem0)
        start_gather(2 * g + 2, g0, sem0)
        drain_slot(2 * g + 1, g1, sem1)
        return carry
    lax.fori_loop(0, n2, edge_pair, 0)
    drain_slot(n_echunk - 1, g0, sem0)
    plsc.subcore_barrier()

    def fin_chunk(c, carry):
        ch = sid + NS * c
        @pl.when(ch < n_fchunk)
        def _():
            base = ch * B
            pltpu.sync_copy(acc.at[pl.ds(base, B)], g0)
            pltpu.sync_copy(zf.at[pl.ds(base, B), cid], g1)
            pltpu.sync_copy(deg.at[pl.ds(base, B)], degb.at[pl.ds(0, B)])

            for gr0 in range(0, B, L):
                dv = degb[pl.ds(gr0, L)]
                rdv = 1.0 / jnp.where(dv == 0.0, 1.0, dv)
                for l in range(L):
                    r = gr0 + l
                    rd = rdv[l]
                    for c8 in range(CH // L):
                        sl = pl.ds(c8 * L, L)
                        zl = gbuf[r, 0, sl] * rd
                        gbuf[r, 0, sl] = zl
                        gbuf[B + r, 0, sl] = gbuf[B + r, 0, sl] - zl

            pltpu.sync_copy(g0, outl.at[pl.ds(base, B), cid])
            pltpu.sync_copy(g1, outh.at[pl.ds(base, B), cid])
        return carry
    lax.fori_loop(0, (n_fchunk + NS - 1) // NS, fin_chunk, 0)


def kernel(Z, edge_index):
    N, C = Z.shape
    E = edge_index.shape[1]
    CH = C // NC
    n_echunk = E // (NS * B)
    n_fchunk = N // B

    zs = Z.reshape(NC * N, 1, CH)
    zf = Z.reshape(N, NC, 1, CH)
    col1 = edge_index[1]
    row2 = edge_index[0].reshape(NS, n_echunk, B)

    body = functools.partial(_sc_body, N, CH, n_echunk, n_fchunk)
    mesh = plsc.VectorSubcoreMesh(core_axis_name="c", subcore_axis_name="s")
    outl, outh = pl.kernel(
        body,
        out_type=(
            jax.ShapeDtypeStruct((N, NC, 1, CH), jnp.float32),
            jax.ShapeDtypeStruct((N, NC, 1, CH), jnp.float32),
        ),
        mesh=mesh,
        scratch_types=(
            pltpu.VMEM((E // NS,), jnp.int32),
            pltpu.VMEM((E // (NS * B), B), jnp.int32),
            pltpu.VMEM((2 * B, 1, CH), jnp.float32),
            pltpu.VMEM((B + L,), jnp.float32),
            pltpu.VMEM((B + L,), jnp.float32),
            pltpu.VMEM_SHARED((N, 1, CH), jnp.float32),
            pltpu.VMEM_SHARED((N,), jnp.float32),
            pltpu.SemaphoreType.DMA,
            pltpu.SemaphoreType.DMA,
        ),
        name="spectral_decomposer_sc",
    )(zs, zf, col1, row2)

    return (outl.reshape(N, C), outh.reshape(N, C))


# speedup vs baseline: 1.2018x; 1.2018x over previous
import functools

import jax
import jax.numpy as jnp
from jax import lax
from jax.experimental import pallas as pl
from jax.experimental.pallas import tpu as pltpu
from jax.experimental.pallas import tpu_sc as plsc

NC = 2
NS = 16
L = 16
B = 80


def _sc_body(N, CH, n_echunk, n_fchunk,
             zs, colb_h, rowb_h, outl, outh,
             colb, rowb, gbuf, onesb, degb,
             acc, deg, gsem0, gsem1, asem0, asem1, dsem0, dsem1):
    cid = lax.axis_index("c")
    sid = lax.axis_index("s")
    zero16 = jnp.zeros((L,), jnp.float32)
    ones16 = jnp.ones((L,), jnp.float32)
    g0 = gbuf.at[pl.ds(0, B)]
    g1 = gbuf.at[pl.ds(B, B)]

    def init_row(r, carry):
        for c8 in range(CH // L):
            gbuf[r, pl.ds(c8 * L, L)] = zero16
        return carry
    lax.fori_loop(0, 2 * B, init_row, 0)

    def init_small(r, carry):
        onesb[pl.ds(r * L, L)] = zero16
        return carry
    lax.fori_loop(0, (B + L) // L, init_small, 0)

    ne = colb.shape[0]
    pltpu.sync_copy(colb_h.at[pl.ds(sid * ne, ne)], colb)
    pltpu.sync_copy(rowb_h.at[sid], rowb)

    def col_fix(i, carry):
        sl = pl.ds(i * L, L)
        colb[sl] = colb[sl] + cid * N
        return carry
    lax.fori_loop(0, ne // L, col_fix, 0)

    def zero_chunk(c, carry):
        ch = sid + NS * c
        @pl.when(ch < N // B)
        def _():
            pltpu.sync_copy(g0, acc.at[pl.ds(ch * B, B)])
            pltpu.sync_copy(onesb.at[pl.ds(0, B)], deg.at[pl.ds(ch * B, B)])
        return carry
    lax.fori_loop(0, (N // B + NS - 1) // NS, zero_chunk, 0)

    def ones_row(r, carry):
        onesb[pl.ds(r * L, L)] = ones16
        return carry
    lax.fori_loop(0, (B + L) // L, ones_row, 0)
    plsc.subcore_barrier()

    def start_gather(k, gb, gsem):
        pltpu.async_copy(zs.at[colb.at[pl.ds(k * B, B)]], gb, gsem)

    def wait_gather(k, gb, gsem):
        pltpu.make_async_copy(zs.at[colb.at[pl.ds(k * B, B)]], gb, gsem).wait()

    def start_scatters(k, gb, asem, dsem):
        pltpu.async_copy(gb, acc.at[rowb.at[k]], asem, add=True)
        pltpu.async_copy(onesb.at[pl.ds(0, B)], deg.at[rowb.at[k]], dsem,
                         add=True)

    def wait_scatters(k, gb, asem, dsem):
        pltpu.make_async_copy(gb, acc.at[rowb.at[k]], asem).wait()
        pltpu.make_async_copy(onesb.at[pl.ds(0, B)], deg.at[rowb.at[k]],
                              dsem).wait()

    start_gather(0, g0, gsem0)
    start_gather(1, g1, gsem1)
    n2 = n_echunk // 2

    def edge_pair(g, carry):
        wait_gather(2 * g, g0, gsem0)
        start_scatters(2 * g, g0, asem0, dsem0)
        wait_gather(2 * g + 1, g1, gsem1)
        start_scatters(2 * g + 1, g1, asem1, dsem1)
        wait_scatters(2 * g, g0, asem0, dsem0)
        start_gather(2 * g + 2, g0, gsem0)
        wait_scatters(2 * g + 1, g1, asem1, dsem1)
        @pl.when(g < n2 - 1)
        def _():
            start_gather(2 * g + 3, g1, gsem1)
        return carry
    lax.fori_loop(0, n2, edge_pair, 0)
    wait_gather(n_echunk - 1, g0, gsem0)
    start_scatters(n_echunk - 1, g0, asem0, dsem0)
    wait_scatters(n_echunk - 1, g0, asem0, dsem0)
    plsc.subcore_barrier()

    def fin_chunk(c, carry):
        ch = sid + NS * c
        @pl.when(ch < n_fchunk)
        def _():
            base = ch * B
            pltpu.sync_copy(acc.at[pl.ds(base, B)], g0)
            pltpu.sync_copy(zs.at[pl.ds(cid * N + base, B)], g1)
            pltpu.sync_copy(deg.at[pl.ds(base, B)], degb.at[pl.ds(0, B)])

            for gr0 in range(0, B, L):
                dv = degb[pl.ds(gr0, L)]
                rdv = 1.0 / jnp.where(dv == 0.0, 1.0, dv)
                for l in range(L):
                    r = gr0 + l
                    rd = rdv[l]
                    for c8 in range(CH // L):
                        sl = pl.ds(c8 * L, L)
                        zl = gbuf[r, sl] * rd
                        gbuf[r, sl] = zl
                        gbuf[B + r, sl] = gbuf[B + r, sl] - zl

            pltpu.sync_copy(g0, outl.at[pl.ds(cid * N + base, B)])
            pltpu.sync_copy(g1, outh.at[pl.ds(cid * N + base, B)])
        return carry
    lax.fori_loop(0, (n_fchunk + NS - 1) // NS, fin_chunk, 0)


def kernel(Z, edge_index):
    N, C = Z.shape
    E = edge_index.shape[1]
    CH = C // NC
    n_echunk = E // (NS * B)
    n_fchunk = N // B

    zs = Z.reshape(N, NC, CH).transpose(1, 0, 2).reshape(NC * N, CH)
    col1 = edge_index[1]
    row2 = edge_index[0].reshape(NS, n_echunk, B)

    body = functools.partial(_sc_body, N, CH, n_echunk, n_fchunk)
    mesh = plsc.VectorSubcoreMesh(core_axis_name="c", subcore_axis_name="s")
    outl, outh = pl.kernel(
        body,
        out_type=(
            jax.ShapeDtypeStruct((NC * N, CH), jnp.float32),
            jax.ShapeDtypeStruct((NC * N, CH), jnp.float32),
        ),
        mesh=mesh,
        scratch_types=(
            pltpu.VMEM((E // NS,), jnp.int32),
            pltpu.VMEM((E // (NS * B), B), jnp.int32),
            pltpu.VMEM((2 * B, CH), jnp.float32),
            pltpu.VMEM((B + L,), jnp.float32),
            pltpu.VMEM((B + L,), jnp.float32),
            pltpu.VMEM_SHARED((N, CH), jnp.float32),
            pltpu.VMEM_SHARED((N,), jnp.float32),
            pltpu.SemaphoreType.DMA,
            pltpu.SemaphoreType.DMA,
            pltpu.SemaphoreType.DMA,
            pltpu.SemaphoreType.DMA,
            pltpu.SemaphoreType.DMA,
            pltpu.SemaphoreType.DMA,
        ),
        name="spectral_decomposer_sc",
    )(zs, col1, row2)

    z_low = outl.reshape(NC, N, CH).transpose(1, 0, 2).reshape(N, C)
    z_high = outh.reshape(NC, N, CH).transpose(1, 0, 2).reshape(N, C)
    return (z_low, z_high)


# speedup vs baseline: 1.3322x; 1.1085x over previous
import functools

import jax
import jax.numpy as jnp
from jax import lax
from jax.experimental import pallas as pl
from jax.experimental.pallas import tpu as pltpu
from jax.experimental.pallas import tpu_sc as plsc

NC = 2
NS = 16
L = 16
B = 80


def _sc_body(N, CH, n_echunk, n_fchunk,
             zs, colb_h, rowb_h, outl, outh,
             colb, rowb, gbuf, onesb, degb,
             acc, deg, gsem0, gsem1, asem0, asem1, dsem0, dsem1):
    cid = lax.axis_index("c")
    sid = lax.axis_index("s")
    zero16 = jnp.zeros((L,), jnp.float32)
    ones16 = jnp.ones((L,), jnp.float32)
    g0 = gbuf.at[pl.ds(0, B)]
    g1 = gbuf.at[pl.ds(B, B)]

    def init_row(r, carry):
        for c8 in range(CH // L):
            gbuf[r, pl.ds(c8 * L, L)] = zero16
        return carry
    lax.fori_loop(0, 2 * B, init_row, 0)

    def init_small(r, carry):
        onesb[pl.ds(r * L, L)] = zero16
        return carry
    lax.fori_loop(0, (B + L) // L, init_small, 0)

    ne = colb.shape[0]
    w = cid * NS + sid
    pltpu.sync_copy(colb_h.at[pl.ds(w * ne, ne)], colb)
    pltpu.sync_copy(rowb_h.at[sid], rowb)

    def zero_chunk(c, carry):
        ch = sid + NS * c
        @pl.when(ch < N // B)
        def _():
            pltpu.sync_copy(g0, acc.at[pl.ds(ch * B, B)])
            pltpu.sync_copy(onesb.at[pl.ds(0, B)], deg.at[pl.ds(ch * B, B)])
        return carry
    lax.fori_loop(0, (N // B + NS - 1) // NS, zero_chunk, 0)

    def ones_row(r, carry):
        onesb[pl.ds(r * L, L)] = ones16
        return carry
    lax.fori_loop(0, (B + L) // L, ones_row, 0)
    plsc.subcore_barrier()

    def start_gather(k, gb, gsem):
        pltpu.async_copy(zs.at[colb.at[pl.ds(k * B, B)]], gb, gsem)

    def wait_gather(k, gb, gsem):
        pltpu.make_async_copy(zs.at[colb.at[pl.ds(k * B, B)]], gb, gsem).wait()

    def start_scatters(k, gb, asem, dsem):
        pltpu.async_copy(gb, acc.at[rowb.at[k]], asem, add=True)
        pltpu.async_copy(onesb.at[pl.ds(0, B)], deg.at[rowb.at[k]], dsem,
                         add=True)

    def wait_scatters(k, gb, asem, dsem):
        pltpu.make_async_copy(gb, acc.at[rowb.at[k]], asem).wait()
        pltpu.make_async_copy(onesb.at[pl.ds(0, B)], deg.at[rowb.at[k]],
                              dsem).wait()

    def drain_slot(k, gb, gsem, asem, dsem):
        wait_gather(k, gb, gsem)
        start_scatters(k, gb, asem, dsem)
        wait_scatters(k, gb, asem, dsem)

    start_gather(0, g0, gsem0)
    n2 = n_echunk // 2

    def edge_pair(g, carry):
        start_gather(2 * g + 1, g1, gsem1)
        drain_slot(2 * g, g0, gsem0, asem0, dsem0)
        start_gather(2 * g + 2, g0, gsem0)
        drain_slot(2 * g + 1, g1, gsem1, asem1, dsem1)
        return carry
    lax.fori_loop(0, n2, edge_pair, 0)
    drain_slot(n_echunk - 1, g0, gsem0, asem0, dsem0)
    plsc.subcore_barrier()

    def fin_chunk(c, carry):
        ch = sid + NS * c
        @pl.when(ch < n_fchunk)
        def _():
            base = ch * B
            pltpu.sync_copy(acc.at[pl.ds(base, B)], g0)
            pltpu.sync_copy(zs.at[pl.ds(cid * N + base, B)], g1)
            pltpu.sync_copy(deg.at[pl.ds(base, B)], degb.at[pl.ds(0, B)])

            for gr0 in range(0, B, L):
                dv = degb[pl.ds(gr0, L)]
                rdv = 1.0 / jnp.where(dv == 0.0, 1.0, dv)
                for l in range(L):
                    r = gr0 + l
                    rd = rdv[l]
                    for c8 in range(CH // L):
                        sl = pl.ds(c8 * L, L)
                        zl = gbuf[r, sl] * rd
                        gbuf[r, sl] = zl
                        gbuf[B + r, sl] = gbuf[B + r, sl] - zl

            pltpu.sync_copy(g0, outl.at[pl.ds(cid * N + base, B)])
            pltpu.sync_copy(g1, outh.at[pl.ds(cid * N + base, B)])
        return carry
    lax.fori_loop(0, (n_fchunk + NS - 1) // NS, fin_chunk, 0)


def kernel(Z, edge_index):
    N, C = Z.shape
    E = edge_index.shape[1]
    CH = C // NC
    n_echunk = E // (NS * B)
    n_fchunk = N // B

    zs = Z.reshape(N, NC, CH).transpose(1, 0, 2).reshape(NC * N, CH)
    col2 = jnp.concatenate([col1 := edge_index[1], col1 + N])
    row2 = edge_index[0].reshape(NS, n_echunk, B)

    body = functools.partial(_sc_body, N, CH, n_echunk, n_fchunk)
    mesh = plsc.VectorSubcoreMesh(core_axis_name="c", subcore_axis_name="s")
    outl, outh = pl.kernel(
        body,
        out_type=(
            jax.ShapeDtypeStruct((NC * N, CH), jnp.float32),
            jax.ShapeDtypeStruct((NC * N, CH), jnp.float32),
        ),
        mesh=mesh,
        scratch_types=(
            pltpu.VMEM((E // NS,), jnp.int32),
            pltpu.VMEM((E // (NS * B), B), jnp.int32),
            pltpu.VMEM((2 * B, CH), jnp.float32),
            pltpu.VMEM((B + L,), jnp.float32),
            pltpu.VMEM((B + L,), jnp.float32),
            pltpu.VMEM_SHARED((N, CH), jnp.float32),
            pltpu.VMEM_SHARED((N,), jnp.float32),
            pltpu.SemaphoreType.DMA,
            pltpu.SemaphoreType.DMA,
            pltpu.SemaphoreType.DMA,
            pltpu.SemaphoreType.DMA,
            pltpu.SemaphoreType.DMA,
            pltpu.SemaphoreType.DMA,
        ),
        name="spectral_decomposer_sc",
    )(zs, col2, row2)

    z_low = outl.reshape(NC, N, CH).transpose(1, 0, 2).reshape(N, C)
    z_high = outh.reshape(NC, N, CH).transpose(1, 0, 2).reshape(N, C)
    return (z_low, z_high)


# speedup vs baseline: 1.3447x; 1.0094x over previous
import functools

import jax
import jax.numpy as jnp
from jax import lax
from jax.experimental import pallas as pl
from jax.experimental.pallas import tpu as pltpu
from jax.experimental.pallas import tpu_sc as plsc

NC = 2
NS = 16
L = 16
B = 80


def _sc_body(N, CH, n_echunk, n_fchunk,
             zs, colb_h, rowb_h, outl, outh,
             colb, rowb, gbuf, onesb, degb,
             acc, deg, gsem0, gsem1, asem0, asem1, dsem0, dsem1):
    cid = lax.axis_index("c")
    sid = lax.axis_index("s")
    zero16 = jnp.zeros((L,), jnp.float32)
    ones16 = jnp.ones((L,), jnp.float32)
    g0 = gbuf.at[pl.ds(0, B)]
    g1 = gbuf.at[pl.ds(B, B)]

    def init_row(r, carry):
        for c8 in range(CH // L):
            gbuf[r, pl.ds(c8 * L, L)] = zero16
        return carry
    lax.fori_loop(0, 2 * B, init_row, 0)

    def init_small(r, carry):
        onesb[pl.ds(r * L, L)] = zero16
        return carry
    lax.fori_loop(0, (B + L) // L, init_small, 0)

    ne = colb.shape[0]
    w = cid * NS + sid
    pltpu.sync_copy(colb_h.at[pl.ds(w * ne, ne)], colb)
    pltpu.sync_copy(rowb_h.at[sid], rowb)

    def zero_chunk(c, carry):
        ch = sid + NS * c
        @pl.when(ch < N // B)
        def _():
            pltpu.sync_copy(g0, acc.at[pl.ds(ch * B, B)])
            pltpu.sync_copy(onesb.at[pl.ds(0, B)], deg.at[pl.ds(ch * B, B)])
        return carry
    lax.fori_loop(0, (N // B + NS - 1) // NS, zero_chunk, 0)

    def ones_row(r, carry):
        onesb[pl.ds(r * L, L)] = ones16
        return carry
    lax.fori_loop(0, (B + L) // L, ones_row, 0)
    plsc.subcore_barrier()

    def start_gather(k, gb, gsem):
        pltpu.async_copy(zs.at[colb.at[pl.ds(k * B, B)]], gb, gsem)

    def wait_gather(k, gb, gsem):
        pltpu.make_async_copy(zs.at[colb.at[pl.ds(k * B, B)]], gb, gsem).wait()

    def start_scatters(k, gb, asem, dsem):
        pltpu.async_copy(gb, acc.at[rowb.at[k]], asem, add=True)
        pltpu.async_copy(onesb.at[pl.ds(0, B)], deg.at[rowb.at[k]], dsem,
                         add=True)

    def wait_scatters(k, gb, asem, dsem):
        pltpu.make_async_copy(gb, acc.at[rowb.at[k]], asem).wait()
        pltpu.make_async_copy(onesb.at[pl.ds(0, B)], deg.at[rowb.at[k]],
                              dsem).wait()

    def drain_slot(k, gb, gsem, asem, dsem):
        wait_gather(k, gb, gsem)
        start_scatters(k, gb, asem, dsem)
        wait_scatters(k, gb, asem, dsem)

    start_gather(0, g0, gsem0)
    n2 = n_echunk // 2

    def edge_pair(g, carry):
        start_gather(2 * g + 1, g1, gsem1)
        drain_slot(2 * g, g0, gsem0, asem0, dsem0)
        start_gather(2 * g + 2, g0, gsem0)
        drain_slot(2 * g + 1, g1, gsem1, asem1, dsem1)
        return carry
    lax.fori_loop(0, n2, edge_pair, 0)
    drain_slot(n_echunk - 1, g0, gsem0, asem0, dsem0)
    plsc.subcore_barrier()

    def fin_chunk(c, carry):
        ch = sid + NS * c
        @pl.when(ch < n_fchunk)
        def _():
            base = ch * B
            pltpu.async_copy(acc.at[pl.ds(base, B)], g0, gsem0)
            pltpu.async_copy(zs.at[pl.ds(cid * N + base, B)], g1, gsem1)
            pltpu.async_copy(deg.at[pl.ds(base, B)], degb.at[pl.ds(0, B)],
                             dsem0)
            pltpu.make_async_copy(acc.at[pl.ds(base, B)], g0, gsem0).wait()
            pltpu.make_async_copy(zs.at[pl.ds(cid * N + base, B)], g1,
                                  gsem1).wait()
            pltpu.make_async_copy(deg.at[pl.ds(base, B)],
                                  degb.at[pl.ds(0, B)], dsem0).wait()

            for gr0 in range(0, B, L):
                dv = degb[pl.ds(gr0, L)]
                rdv = 1.0 / jnp.where(dv == 0.0, 1.0, dv)
                for l in range(L):
                    r = gr0 + l
                    rd = rdv[l]
                    for c8 in range(CH // L):
                        sl = pl.ds(c8 * L, L)
                        zl = gbuf[r, sl] * rd
                        gbuf[r, sl] = zl
                        gbuf[B + r, sl] = gbuf[B + r, sl] - zl

            pltpu.async_copy(g0, outl.at[pl.ds(cid * N + base, B)], asem0)
            pltpu.async_copy(g1, outh.at[pl.ds(cid * N + base, B)], asem1)
            pltpu.make_async_copy(g0, outl.at[pl.ds(cid * N + base, B)],
                                  asem0).wait()
            pltpu.make_async_copy(g1, outh.at[pl.ds(cid * N + base, B)],
                                  asem1).wait()
        return carry
    lax.fori_loop(0, (n_fchunk + NS - 1) // NS, fin_chunk, 0)


def kernel(Z, edge_index):
    N, C = Z.shape
    E = edge_index.shape[1]
    CH = C // NC
    n_echunk = E // (NS * B)
    n_fchunk = N // B

    zs = Z.reshape(N, NC, CH).transpose(1, 0, 2).reshape(NC * N, CH)
    col2 = jnp.concatenate([col1 := edge_index[1], col1 + N])
    row2 = edge_index[0].reshape(NS, n_echunk, B)

    body = functools.partial(_sc_body, N, CH, n_echunk, n_fchunk)
    mesh = plsc.VectorSubcoreMesh(core_axis_name="c", subcore_axis_name="s")
    outl, outh = pl.kernel(
        body,
        out_type=(
            jax.ShapeDtypeStruct((NC * N, CH), jnp.float32),
            jax.ShapeDtypeStruct((NC * N, CH), jnp.float32),
        ),
        mesh=mesh,
        scratch_types=(
            pltpu.VMEM((E // NS,), jnp.int32),
            pltpu.VMEM((E // (NS * B), B), jnp.int32),
            pltpu.VMEM((2 * B, CH), jnp.float32),
            pltpu.VMEM((B + L,), jnp.float32),
            pltpu.VMEM((B + L,), jnp.float32),
            pltpu.VMEM_SHARED((N, CH), jnp.float32),
            pltpu.VMEM_SHARED((N,), jnp.float32),
            pltpu.SemaphoreType.DMA,
            pltpu.SemaphoreType.DMA,
            pltpu.SemaphoreType.DMA,
            pltpu.SemaphoreType.DMA,
            pltpu.SemaphoreType.DMA,
            pltpu.SemaphoreType.DMA,
        ),
        name="spectral_decomposer_sc",
    )(zs, col2, row2)

    z_low = outl.reshape(NC, N, CH).transpose(1, 0, 2).reshape(N, C)
    z_high = outh.reshape(NC, N, CH).transpose(1, 0, 2).reshape(N, C)
    return (z_low, z_high)


# speedup vs baseline: 1.3930x; 1.0359x over previous
import functools

import jax
import jax.numpy as jnp
from jax import lax
from jax.experimental import pallas as pl
from jax.experimental.pallas import tpu as pltpu
from jax.experimental.pallas import tpu_sc as plsc

NC = 2
NS = 16
L = 16
B = 104
NPAD = 88
BZ = 80


def _sc_body(N, CH, n_echunk, n_fchunk,
             zs, colb_h, rowb_h, outl, outh,
             colb, rowb, gbuf, onesb, degb,
             acc, deg, gsem0, gsem1, asem0, asem1, dsem0, dsem1):
    cid = lax.axis_index("c")
    sid = lax.axis_index("s")
    zero16 = jnp.zeros((L,), jnp.float32)
    ones16 = jnp.ones((L,), jnp.float32)
    g0 = gbuf.at[pl.ds(0, B)]
    g1 = gbuf.at[pl.ds(B, B)]
    g0f = gbuf.at[pl.ds(0, BZ)]
    g1f = gbuf.at[pl.ds(B, BZ)]

    def init_row(r, carry):
        for c8 in range(CH // L):
            gbuf[r, pl.ds(c8 * L, L)] = zero16
        return carry
    lax.fori_loop(0, 2 * B, init_row, 0)

    def init_small(r, carry):
        onesb[pl.ds(r * L, L)] = zero16
        return carry
    lax.fori_loop(0, (B + L) // L, init_small, 0)

    ne = colb.shape[0]
    w = cid * NS + sid
    pltpu.sync_copy(colb_h.at[pl.ds(w * ne, ne)], colb)
    pltpu.sync_copy(rowb_h.at[sid], rowb)

    gz = gbuf.at[pl.ds(0, BZ)]

    def zero_chunk(c, carry):
        ch = sid + NS * c
        @pl.when(ch < N // BZ)
        def _():
            pltpu.sync_copy(gz, acc.at[pl.ds(ch * BZ, BZ)])
            pltpu.sync_copy(onesb.at[pl.ds(0, BZ)], deg.at[pl.ds(ch * BZ, BZ)])
        return carry
    lax.fori_loop(0, (N // BZ + NS - 1) // NS, zero_chunk, 0)

    def ones_row(r, carry):
        onesb[pl.ds(r * L, L)] = ones16
        return carry
    lax.fori_loop(0, (B + L) // L, ones_row, 0)
    plsc.subcore_barrier()

    def start_gather(k, gb, gsem):
        pltpu.async_copy(zs.at[colb.at[pl.ds(k * B, B)]], gb, gsem)

    def wait_gather(k, gb, gsem):
        pltpu.make_async_copy(zs.at[colb.at[pl.ds(k * B, B)]], gb, gsem).wait()

    def start_scatters(k, gb, asem, dsem):
        pltpu.async_copy(gb, acc.at[rowb.at[k]], asem, add=True)
        pltpu.async_copy(onesb.at[pl.ds(0, B)], deg.at[rowb.at[k]], dsem,
                         add=True)

    def wait_scatters(k, gb, asem, dsem):
        pltpu.make_async_copy(gb, acc.at[rowb.at[k]], asem).wait()
        pltpu.make_async_copy(onesb.at[pl.ds(0, B)], deg.at[rowb.at[k]],
                              dsem).wait()

    def drain_slot(k, gb, gsem, asem, dsem):
        wait_gather(k, gb, gsem)
        start_scatters(k, gb, asem, dsem)
        wait_scatters(k, gb, asem, dsem)

    start_gather(0, g0, gsem0)
    n2 = n_echunk // 2

    def edge_pair(g, carry):
        start_gather(2 * g + 1, g1, gsem1)
        drain_slot(2 * g, g0, gsem0, asem0, dsem0)
        start_gather(2 * g + 2, g0, gsem0)
        drain_slot(2 * g + 1, g1, gsem1, asem1, dsem1)
        return carry
    lax.fori_loop(0, n2, edge_pair, 0)
    drain_slot(n_echunk - 1, g0, gsem0, asem0, dsem0)
    plsc.subcore_barrier()

    BF = BZ
    def fin_chunk(c, carry):
        ch = sid + NS * c
        @pl.when(ch < n_fchunk)
        def _():
            base = ch * BF
            pltpu.async_copy(acc.at[pl.ds(base, BF)], g0f, gsem0)
            pltpu.async_copy(zs.at[pl.ds(cid * N + base, BF)], g1f, gsem1)
            pltpu.async_copy(deg.at[pl.ds(base, BF)], degb.at[pl.ds(0, BF)],
                             dsem0)
            pltpu.make_async_copy(acc.at[pl.ds(base, BF)], g0f, gsem0).wait()
            pltpu.make_async_copy(zs.at[pl.ds(cid * N + base, BF)], g1f,
                                  gsem1).wait()
            pltpu.make_async_copy(deg.at[pl.ds(base, BF)],
                                  degb.at[pl.ds(0, BF)], dsem0).wait()

            for gr0 in range(0, BF, L):
                dv = degb[pl.ds(gr0, L)]
                rdv = 1.0 / jnp.where(dv == 0.0, 1.0, dv)
                for l in range(L):
                    r = gr0 + l
                    rd = rdv[l]
                    for c8 in range(CH // L):
                        sl = pl.ds(c8 * L, L)
                        zl = gbuf[r, sl] * rd
                        gbuf[r, sl] = zl
                        gbuf[B + r, sl] = gbuf[B + r, sl] - zl

            pltpu.async_copy(g0f, outl.at[pl.ds(cid * N + base, BF)], asem0)
            pltpu.async_copy(g1f, outh.at[pl.ds(cid * N + base, BF)], asem1)
            pltpu.make_async_copy(g0f, outl.at[pl.ds(cid * N + base, BF)],
                                  asem0).wait()
            pltpu.make_async_copy(g1f, outh.at[pl.ds(cid * N + base, BF)],
                                  asem1).wait()
        return carry
    lax.fori_loop(0, (n_fchunk + NS - 1) // NS, fin_chunk, 0)


def kernel(Z, edge_index):
    N, C = Z.shape
    E = edge_index.shape[1]
    CH = C // NC
    ne_p = E // NS + NPAD
    n_echunk = ne_p // B
    n_fchunk = N // BZ

    zs = Z.reshape(N, NC, CH).transpose(1, 0, 2).reshape(NC * N, CH)
    pad_col = jnp.broadcast_to((jnp.arange(NPAD, dtype=jnp.int32) * 113) % N,
                               (NS, NPAD))
    pad_row = jnp.broadcast_to(N + (jnp.arange(NPAD, dtype=jnp.int32) % 16),
                               (NS, NPAD))
    colp = jnp.concatenate([edge_index[1].reshape(NS, E // NS), pad_col],
                           axis=1).reshape(-1)
    col2 = jnp.concatenate([colp, colp + N])
    row2 = jnp.concatenate([edge_index[0].reshape(NS, E // NS), pad_row],
                           axis=1).reshape(NS, n_echunk, B)

    body = functools.partial(_sc_body, N, CH, n_echunk, n_fchunk)
    mesh = plsc.VectorSubcoreMesh(core_axis_name="c", subcore_axis_name="s")
    outl, outh = pl.kernel(
        body,
        out_type=(
            jax.ShapeDtypeStruct((NC * N, CH), jnp.float32),
            jax.ShapeDtypeStruct((NC * N, CH), jnp.float32),
        ),
        mesh=mesh,
        scratch_types=(
            pltpu.VMEM((ne_p,), jnp.int32),
            pltpu.VMEM((n_echunk, B), jnp.int32),
            pltpu.VMEM((2 * B, CH), jnp.float32),
            pltpu.VMEM((B + L,), jnp.float32),
            pltpu.VMEM((B + L,), jnp.float32),
            pltpu.VMEM_SHARED((N + 16, CH), jnp.float32),
            pltpu.VMEM_SHARED((N + 16,), jnp.float32),
            pltpu.SemaphoreType.DMA,
            pltpu.SemaphoreType.DMA,
            pltpu.SemaphoreType.DMA,
            pltpu.SemaphoreType.DMA,
            pltpu.SemaphoreType.DMA,
            pltpu.SemaphoreType.DMA,
        ),
        name="spectral_decomposer_sc",
    )(zs, col2, row2)

    z_low = outl.reshape(NC, N, CH).transpose(1, 0, 2).reshape(N, C)
    z_high = outh.reshape(NC, N, CH).transpose(1, 0, 2).reshape(N, C)
    return (z_low, z_high)
